# baseline (speedup 1.0000x reference)
import dataclasses
import functools

import jax
import jax.numpy as jnp
from jax import lax
from jax.experimental import pallas as pl
from jax.experimental.pallas import tpu as pltpu
from jax.experimental.pallas import tpu_sc as plsc

NC = 2
NS = 16
NW = NC * NS
N = 16384
D = 512
C = 1000
CP = 1024
L = 16
RPW = N // NW
CHUNK = 256
NCHUNK = N // CHUNK
MOM = 0.9


def _segment_sums_sc(zt, y):
    mesh = plsc.VectorSubcoreMesh(core_axis_name="c", subcore_axis_name="s")
    cp = pltpu.CompilerParams()
    if "needs_layout_passes" in pltpu.CompilerParams.__dataclass_fields__:
        cp = dataclasses.replace(cp, needs_layout_passes=False)
    if "use_tc_tiling_on_sc" in pltpu.CompilerParams.__dataclass_fields__:
        cp = dataclasses.replace(cp, use_tc_tiling_on_sc=False)

    @functools.partial(
        pl.kernel,
        compiler_params=cp,
        out_type=[
            jax.ShapeDtypeStruct((NW, 8, CP * L // 8), jnp.float32),
            jax.ShapeDtypeStruct((NW, 8, CP * L // 8), jnp.float32),
        ],
        mesh=mesh,
        scratch_types=[
            pltpu.VMEM((CP, L), jnp.float32),
            pltpu.VMEM((CP, L), jnp.float32),
            pltpu.VMEM((8, CP * L // 8), jnp.float32),
            pltpu.VMEM((CHUNK * L,), jnp.float32),
            pltpu.VMEM((CHUNK,), jnp.int32),
            pltpu.VMEM((RPW,), jnp.int32),
        ],
    )
    def body(zt_hbm, y_hbm, sums_hbm, cnt_hbm, acc, cnt, stage, z_v, y_v,
             y2_v):
        c = lax.axis_index("c")
        s = lax.axis_index("s")
        wid = s * NC + c

        zeros16 = jnp.zeros((L,), jnp.float32)
        iota16 = lax.iota(jnp.int32, L)
        ones16 = jnp.ones((L,), jnp.float32)

        @pl.loop(0, CP)
        def _(i):
            acc[i, :] = zeros16
            cnt[i, :] = zeros16

        pltpu.sync_copy(y_hbm.at[pl.ds(wid * RPW, RPW)], y2_v)

        @pl.loop(0, RPW, step=L)
        def _(g):
            y16 = y2_v[pl.ds(g, L)]
            plsc.addupdate_scatter(cnt, [y16, iota16], ones16)

        stripe0 = wid * (N * L)

        @pl.loop(0, NCHUNK)
        def _(i):
            r0 = i * CHUNK
            pltpu.sync_copy(y_hbm.at[pl.ds(r0, CHUNK)], y_v)
            pltpu.sync_copy(zt_hbm.at[pl.ds(stripe0 + r0 * L, CHUNK * L)], z_v)

            @pl.loop(0, CHUNK, step=8)
            def _(g):
                for r8 in range(8):
                    r = g + r8
                    yb = plsc.load_gather(y_v, [jnp.full((L,), r, jnp.int32)])
                    plsc.addupdate_scatter(acc, [yb, iota16],
                                           z_v[pl.ds(r * L, L)])

        @pl.loop(0, 8)
        def _(j):
            @pl.loop(0, CP // 8)
            def _(k):
                stage[j, pl.ds(k * L, L)] = acc[j * (CP // 8) + k, :]

        pltpu.sync_copy(stage, sums_hbm.at[wid])

        @pl.loop(0, 8)
        def _(j):
            @pl.loop(0, CP // 8)
            def _(k):
                stage[j, pl.ds(k * L, L)] = cnt[j * (CP // 8) + k, :]

        pltpu.sync_copy(stage, cnt_hbm.at[wid])

    return body(zt, y)


def _epilogue_body(sums_ref, cnt_ref, proto_ref, counts_ref,
                   proto_out_ref, counts_out_ref):
    sums = sums_ref[...][:C]
    cnt3 = cnt_ref[...].reshape(NW, CP, L)
    cnt_all = jnp.sum(cnt3, axis=0)
    cnt = jnp.sum(cnt_all, axis=1, keepdims=True)[:C]
    proto = proto_ref[...]

    cnt_safe = jnp.where(cnt > 0, cnt, 1.0)
    z_mean = sums / cnt_safe
    n1 = jnp.sqrt(jnp.sum(z_mean * z_mean, axis=1, keepdims=True))
    z_mean_n = z_mean / jnp.maximum(n1, 1e-12)
    new = MOM * proto + (1.0 - MOM) * z_mean_n
    n2 = jnp.sqrt(jnp.sum(new * new, axis=1, keepdims=True))
    new_n = new / jnp.maximum(n2, 1e-12)
    proto_out_ref[...] = jnp.where(cnt > 0, new_n, proto)
    counts_out_ref[...] = counts_ref[...] + cnt


def kernel(z, y, proto, counts):
    y = y.reshape(-1).astype(jnp.int32)
    zt = z.reshape(N, NW, L).transpose(1, 0, 2).reshape(NW * N * L)
    sums3, cnt3 = _segment_sums_sc(zt, y)
    sums = sums3.reshape(NW, CP, L).transpose(1, 0, 2).reshape(CP, D)

    proto_new, counts_new = pl.pallas_call(
        _epilogue_body,
        out_shape=[
            jax.ShapeDtypeStruct((C, D), jnp.float32),
            jax.ShapeDtypeStruct((C, 1), jnp.float32),
        ],
    )(sums, cnt3.reshape(NW, CP * L), proto, counts.reshape(C, 1))

    return proto_new, counts_new.reshape(-1)


# speedup vs baseline: 2.8862x; 2.8862x over previous
import dataclasses
import functools

import jax
import jax.numpy as jnp
from jax import lax
from jax.experimental import pallas as pl
from jax.experimental.pallas import tpu as pltpu
from jax.experimental.pallas import tpu_sc as plsc

NC = 2
NS = 16
NW = NC * NS
N = 16384
D = 512
C = 1000
CP = 1024
L = 16
MOM = 0.9

NG = 8
NB = 4
CB = D // NB
RPG = N // NG
ZR = 8
NCH = RPG // ZR
YC = 256
RPW = N // NW


def _sc_compiler_params(use_tc_tiling):
    cp = pltpu.CompilerParams()
    fields = pltpu.CompilerParams.__dataclass_fields__
    if "needs_layout_passes" in fields:
        cp = dataclasses.replace(cp, needs_layout_passes=False)
    if not use_tc_tiling and "use_tc_tiling_on_sc" in fields:
        cp = dataclasses.replace(cp, use_tc_tiling_on_sc=False)
    return cp


def _segment_sums_sc(z, y):
    mesh = plsc.VectorSubcoreMesh(core_axis_name="c", subcore_axis_name="s")

    @functools.partial(
        pl.kernel,
        compiler_params=_sc_compiler_params(use_tc_tiling=True),
        out_type=jax.ShapeDtypeStruct((NW, C, CB), jnp.float32),
        mesh=mesh,
        scratch_types=[
            pltpu.VMEM((C, CB), jnp.float32),
            pltpu.VMEM((ZR, CB), jnp.float32),
            pltpu.VMEM((ZR, CB), jnp.float32),
            pltpu.VMEM((YC,), jnp.int32),
            pltpu.SemaphoreType.DMA,
            pltpu.SemaphoreType.DMA,
        ],
    )
    def body(z_hbm, y_hbm, out_hbm, acc, z0, z1, y_v, sem0, sem1):
        c = lax.axis_index("c")
        s = lax.axis_index("s")
        wid = s * NC + c
        g = wid // NB
        b = wid % NB
        rowbase = g * RPG
        col0 = b * CB

        zeros16 = jnp.zeros((L,), jnp.float32)
        iotas = [lax.iota(jnp.int32, L) + j * L for j in range(CB // L)]

        @pl.loop(0, C)
        def _(i):
            for j in range(CB // L):
                acc[i, pl.ds(j * L, L)] = zeros16

        def z_copy(buf, sem, ch):
            return pltpu.make_async_copy(
                z_hbm.at[pl.ds(rowbase + ch * ZR, ZR), pl.ds(col0, CB)],
                buf, sem)

        def process(buf, ch):
            for i in range(ZR):
                loc = (ch % (YC // ZR)) * ZR + i
                yb = plsc.load_gather(y_v, [jnp.full((L,), loc, jnp.int32)])
                for j in range(CB // L):
                    plsc.addupdate_scatter(acc, [yb, iotas[j]],
                                           buf[i, pl.ds(j * L, L)])

        z_copy(z0, sem0, 0).start()
        z_copy(z1, sem1, 1).start()

        @pl.loop(0, NCH // 2)
        def _(h):
            c0 = 2 * h
            c1 = 2 * h + 1

            @pl.when(c0 % (YC // ZR) == 0)
            def _():
                pltpu.sync_copy(
                    y_hbm.at[pl.ds(rowbase + (c0 // (YC // ZR)) * YC, YC)],
                    y_v)

            z_copy(z0, sem0, c0).wait()
            process(z0, c0)

            @pl.when(c0 + 2 < NCH)
            def _():
                z_copy(z0, sem0, c0 + 2).start()

            z_copy(z1, sem1, c1).wait()
            process(z1, c1)

            @pl.when(c1 + 2 < NCH)
            def _():
                z_copy(z1, sem1, c1 + 2).start()

        pltpu.sync_copy(acc, out_hbm.at[wid])

    return body(z, y)


def _segment_counts_sc(y):
    mesh = plsc.VectorSubcoreMesh(core_axis_name="c", subcore_axis_name="s")

    @functools.partial(
        pl.kernel,
        compiler_params=_sc_compiler_params(use_tc_tiling=False),
        out_type=jax.ShapeDtypeStruct((NW, 8, CP * L // 8), jnp.float32),
        mesh=mesh,
        scratch_types=[
            pltpu.VMEM((CP, L), jnp.float32),
            pltpu.VMEM((8, CP * L // 8), jnp.float32),
            pltpu.VMEM((RPW,), jnp.int32),
        ],
    )
    def body(y_hbm, cnt_hbm, cnt, stage, y_v):
        c = lax.axis_index("c")
        s = lax.axis_index("s")
        wid = s * NC + c

        zeros16 = jnp.zeros((L,), jnp.float32)
        iota16 = lax.iota(jnp.int32, L)
        ones16 = jnp.ones((L,), jnp.float32)

        @pl.loop(0, CP)
        def _(i):
            cnt[i, :] = zeros16

        pltpu.sync_copy(y_hbm.at[pl.ds(wid * RPW, RPW)], y_v)

        @pl.loop(0, RPW, step=L)
        def _(g):
            y16 = y_v[pl.ds(g, L)]
            plsc.addupdate_scatter(cnt, [y16, iota16], ones16)

        @pl.loop(0, 8)
        def _(j):
            @pl.loop(0, CP // 8)
            def _(k):
                stage[j, pl.ds(k * L, L)] = cnt[j * (CP // 8) + k, :]

        pltpu.sync_copy(stage, cnt_hbm.at[wid])

    return body(y)


def _epilogue_body(sums_ref, cnt_ref, proto_ref, counts_ref,
                   proto_out_ref, counts_out_ref):
    x = sums_ref[...].reshape(NG, NB, C, CB)
    sblocks = jnp.sum(x, axis=0)
    sums = jnp.concatenate([sblocks[i] for i in range(NB)], axis=1)

    cnt3 = cnt_ref[...].reshape(NW, CP, L)
    cnt_all = jnp.sum(cnt3, axis=0)
    cnt = jnp.sum(cnt_all, axis=1, keepdims=True)[:C]
    proto = proto_ref[...]

    cnt_safe = jnp.where(cnt > 0, cnt, 1.0)
    z_mean = sums / cnt_safe
    n1 = jnp.sqrt(jnp.sum(z_mean * z_mean, axis=1, keepdims=True))
    z_mean_n = z_mean / jnp.maximum(n1, 1e-12)
    new = MOM * proto + (1.0 - MOM) * z_mean_n
    n2 = jnp.sqrt(jnp.sum(new * new, axis=1, keepdims=True))
    new_n = new / jnp.maximum(n2, 1e-12)
    proto_out_ref[...] = jnp.where(cnt > 0, new_n, proto)
    counts_out_ref[...] = counts_ref[...] + cnt


def kernel(z, y, proto, counts):
    y = y.reshape(-1).astype(jnp.int32)
    sums3 = _segment_sums_sc(z, y)
    cnt3 = _segment_counts_sc(y)

    proto_new, counts_new = pl.pallas_call(
        _epilogue_body,
        out_shape=[
            jax.ShapeDtypeStruct((C, D), jnp.float32),
            jax.ShapeDtypeStruct((C, 1), jnp.float32),
        ],
    )(sums3, cnt3.reshape(NW, CP * L), proto, counts.reshape(C, 1))

    return proto_new, counts_new.reshape(-1)
